# Initial kernel scaffold; baseline (speedup 1.0000x reference)
import jax
import jax.numpy as jnp
from jax.experimental import pallas as pl


def kernel(x, edge_index, W, att, bias):
    raise NotImplementedError("write your pallas kernel here")


# speedup vs baseline: 26.1290x; 26.1290x over previous
import functools

import jax
import jax.numpy as jnp
from jax import lax
from jax.experimental import pallas as pl
from jax.experimental.pallas import tpu as pltpu
from jax.experimental.pallas import tpu_sc as plsc

N = 10000
E = 160000
IN = 256
HEADS = 4
OUT = 128
NEG = 0.2

N_TC = 10240
BE = 64
NB = 167
E_PT = NB * BE
E_PAD = 16 * E_PT
NB_A = 2 * NB
E_REAL = E + N
ROWS_PT = N // 16
SHIFT = 16384


def _tc_body(x_ref, w_ref, ati_ref, atj_ref, h_ref, aij_ref):
    xb = x_ref[...]
    h = jnp.dot(xb, w_ref[...], preferred_element_type=jnp.float32)
    h_ref[...] = h
    h4 = h.reshape(xb.shape[0], HEADS, OUT)
    ai = (h4 * ati_ref[...][:HEADS][None, :, :]).sum(-1)
    aj = (h4 * atj_ref[...][:HEADS][None, :, :]).sum(-1)
    pad = jnp.zeros((xb.shape[0], 128 - 2 * HEADS), dtype=jnp.float32)
    aij_ref[...] = jnp.concatenate([ai, aj, pad], axis=1)


def _tc_project(x_pad, W, att_i8, att_j8):
    nb = 1024
    grid = N_TC // nb
    return pl.pallas_call(
        _tc_body,
        grid=(grid,),
        in_specs=[
            pl.BlockSpec((nb, IN), lambda i: (i, 0)),
            pl.BlockSpec((IN, HEADS * OUT), lambda i: (0, 0)),
            pl.BlockSpec((8, 128), lambda i: (0, 0)),
            pl.BlockSpec((8, 128), lambda i: (0, 0)),
        ],
        out_specs=[
            pl.BlockSpec((nb, HEADS * OUT), lambda i: (i, 0)),
            pl.BlockSpec((nb, 128), lambda i: (i, 0)),
        ],
        out_shape=[
            jax.ShapeDtypeStruct((N_TC, HEADS * OUT), jnp.float32),
            jax.ShapeDtypeStruct((N_TC, 128), jnp.float32),
        ],
    )(x_pad, W, att_i8, att_j8)


def _sc_body(hflat, ai_hbm, aj_hbm, m_hbm, rc_flat, zden, out_hbm, ex_hbm,
             den_sp, out_sp, sem, sem2):
    c = lax.axis_index("c")
    s = lax.axis_index("s")
    iota = lax.iota(jnp.int32, 16)

    def region_a(ai_t, aj_t, rc_p, st64, ri64, m_v):
        p = s // 2
        hl = s % 2
        hh = 2 * c + hl
        pltpu.sync_copy(ai_hbm.at[hh], ai_t)
        pltpu.sync_copy(aj_hbm.at[hh], aj_t)
        pltpu.sync_copy(m_hbm.at[c], m_v)
        pltpu.sync_copy(rc_flat.at[pl.ds(p * 2 * E_PT, 2 * E_PT)], rc_p)

        @pl.when(s < 10)
        def _():
            pltpu.sync_copy(zden, den_sp.at[pl.ds(s * 2000, 2000)])

        plsc.subcore_barrier()

        def pa_body(b, carry):
            for g in range(4):
                sl = pl.ds(g * 16, 16)
                rc = rc_p[pl.ds(b * BE + g * 16, 16)]
                r = lax.shift_right_logical(rc, 14)
                cc = lax.bitwise_and(rc, SHIFT - 1)
                gid = jnp.full((16,), p * 2 * E_PT + b * BE + g * 16,
                               jnp.int32) + iota
                live = gid < E_REAL
                air = plsc.load_gather(ai_t, [r])
                ajc = plsc.load_gather(aj_t, [cc])
                sv = air + ajc
                alpha = jnp.where(sv > 0, sv, NEG * sv)
                bb = air + m_v[pl.ds(hl * 16, 16)]
                bp = jnp.where(bb > 0, bb, NEG * bb)
                ex = jnp.where(live, jnp.exp(alpha - bp), 0.0)
                st64[sl] = ex
                ri64[sl] = r + jnp.full((16,), hl * N, jnp.int32)
            pltpu.sync_copy(st64, den_sp.at[ri64], add=True)
            pltpu.sync_copy(
                st64,
                ex_hbm.at[pl.ds(hh * E_PAD + p * 2 * E_PT + b * BE, BE)])
            return carry

        lax.fori_loop(0, NB_A, pa_body, 0)

    pl.run_scoped(
        region_a,
        ai_t=pltpu.VMEM((N,), jnp.float32),
        aj_t=pltpu.VMEM((N,), jnp.float32),
        rc_p=pltpu.VMEM((2 * E_PT,), jnp.int32),
        st64=pltpu.VMEM((BE,), jnp.float32),
        ri64=pltpu.VMEM((BE,), jnp.int32),
        m_v=pltpu.VMEM((32,), jnp.float32),
    )
    plsc.subcore_barrier()

    def region_b(rc_t, rows_v, exb, w64, d64, cidx, ridx, didx):
        pltpu.sync_copy(rc_flat.at[pl.ds(s * E_PT, E_PT)], rc_t)

        def zero_j(j, carry2):
            for q in range(8):
                rows_v[j, pl.ds(q * 16, 16)] = jnp.zeros((16,), jnp.float32)
            return carry2

        for h in range(2):
            hh = 2 * c + h
            lax.fori_loop(0, BE, zero_j, 0)
            for k in range(9):
                pltpu.sync_copy(
                    rows_v, out_sp.at[pl.ds(s * ROWS_PT + k * BE, BE)])
            pltpu.sync_copy(
                rows_v.at[pl.ds(0, ROWS_PT - 9 * BE)],
                out_sp.at[pl.ds(s * ROWS_PT + 9 * BE, ROWS_PT - 9 * BE)])
            plsc.subcore_barrier()

            def pb_body(b, carry, h=h, hh=hh):
                for g in range(4):
                    sl = pl.ds(g * 16, 16)
                    rc = rc_t[pl.ds(b * BE + g * 16, 16)]
                    r = lax.shift_right_logical(rc, 14)
                    cc = lax.bitwise_and(rc, SHIFT - 1)
                    cidx[sl] = cc + jnp.full((16,), hh * N, jnp.int32)
                    ridx[sl] = r
                    didx[sl] = r + jnp.full((16,), h * N, jnp.int32)
                cpr = pltpu.async_copy(hflat.at[cidx], rows_v, sem)
                cpe = pltpu.async_copy(
                    ex_hbm.at[pl.ds(hh * E_PAD + s * E_PT + b * BE, BE)],
                    exb, sem2)
                pltpu.sync_copy(den_sp.at[didx], d64)
                cpe.wait()
                for g in range(4):
                    sl = pl.ds(g * 16, 16)
                    w64[sl] = exb[sl] / (d64[sl] + 1e-10)
                cpr.wait()

                def scale_j(j, carry2):
                    wv = plsc.load_gather(
                        w64, [jnp.full((16,), j, jnp.int32)])
                    for q in range(8):
                        rows_v[j, pl.ds(q * 16, 16)] = (
                            rows_v[j, pl.ds(q * 16, 16)] * wv)
                    return carry2

                lax.fori_loop(0, BE, scale_j, 0)
                pltpu.sync_copy(rows_v, out_sp.at[ridx], add=True)
                return carry

            lax.fori_loop(0, NB, pb_body, 0)
            plsc.subcore_barrier()
            pltpu.sync_copy(
                out_sp.at[pl.ds(s * ROWS_PT, ROWS_PT)],
                out_hbm.at[pl.ds(hh * N + s * ROWS_PT, ROWS_PT)])
            plsc.subcore_barrier()

    pl.run_scoped(
        region_b,
        rc_t=pltpu.VMEM((E_PT,), jnp.int32),
        rows_v=pltpu.VMEM((BE, 128), jnp.float32),
        exb=pltpu.VMEM((BE,), jnp.float32),
        w64=pltpu.VMEM((BE,), jnp.float32),
        d64=pltpu.VMEM((BE,), jnp.float32),
        cidx=pltpu.VMEM((BE,), jnp.int32),
        ridx=pltpu.VMEM((BE,), jnp.int32),
        didx=pltpu.VMEM((BE,), jnp.int32),
    )


def _sc_run(hflat, ai_T, aj_T, Msc, rc_flat, zden):
    mesh = plsc.VectorSubcoreMesh(core_axis_name="c", subcore_axis_name="s")
    kern = functools.partial(
        pl.kernel,
        mesh=mesh,
        compiler_params=pltpu.CompilerParams(
            needs_layout_passes=False, use_tc_tiling_on_sc=False),
        out_type=(
            jax.ShapeDtypeStruct((HEADS * N, 128), jnp.float32),
            jax.ShapeDtypeStruct((HEADS * E_PAD,), jnp.float32),
        ),
        scratch_types=[
            pltpu.VMEM_SHARED((2 * N,), jnp.float32),
            pltpu.VMEM_SHARED((N, 128), jnp.float32),
            pltpu.SemaphoreType.DMA,
            pltpu.SemaphoreType.DMA,
        ],
    )(_sc_body)
    return kern(hflat, ai_T, aj_T, Msc, rc_flat, zden)


@jax.jit
def kernel(x, edge_index, W, att, bias):
    x_pad = jnp.concatenate(
        [x, jnp.zeros((N_TC - N, IN), jnp.float32)], axis=0)
    att2 = att[0]
    att_i8 = jnp.zeros((8, 128), jnp.float32).at[:HEADS].set(att2[:, :OUT])
    att_j8 = jnp.zeros((8, 128), jnp.float32).at[:HEADS].set(att2[:, OUT:])

    h2d, aij = _tc_project(x_pad, W, att_i8, att_j8)
    h2d = h2d[:N]
    aij = aij[:N]

    hflat = h2d.reshape(N, HEADS, OUT).transpose(1, 0, 2) \
               .reshape(HEADS * N, 128)
    ai_T = aij[:, :HEADS].T
    aj_T = aij[:, HEADS:2 * HEADS].T
    M4 = jnp.max(aj_T, axis=1)
    Msc = jnp.broadcast_to(M4.reshape(2, 2, 1), (2, 2, 16)) \
             .reshape(2, 32).astype(jnp.float32)

    loops = jnp.arange(N, dtype=edge_index.dtype)
    ei = jnp.concatenate([edge_index, jnp.stack([loops, loops])], axis=1)
    rc = ei[0] * SHIFT + ei[1]
    rc_flat = jnp.concatenate(
        [rc, jnp.zeros((E_PAD - E_REAL,), rc.dtype)])

    zden = jnp.zeros((2000,), jnp.float32)

    out_flat, _ = _sc_run(hflat, ai_T, aj_T, Msc, rc_flat, zden)

    out = out_flat.reshape(HEADS, N, 128).transpose(1, 0, 2)
    return out.reshape(N, HEADS * OUT) + bias


# speedup vs baseline: 38.0859x; 1.4576x over previous
import functools

import jax
import jax.numpy as jnp
from jax import lax
from jax.experimental import pallas as pl
from jax.experimental.pallas import tpu as pltpu
from jax.experimental.pallas import tpu_sc as plsc

N = 10000
E = 160000
IN = 256
HEADS = 4
OUT = 128
NEG = 0.2

N_TC = 10240
BE = 64
NB = 167
E_PT = NB * BE
E_PAD = 16 * E_PT
NB_A = 2 * NB
E_REAL = E + N
ROWS_PT = N // 16
SHIFT = 16384


def _tc_body(x_ref, w_ref, ati_ref, atj_ref, h_ref, aij_ref):
    xb = x_ref[...]
    h = jnp.dot(xb, w_ref[...], preferred_element_type=jnp.float32)
    h_ref[...] = h
    h4 = h.reshape(xb.shape[0], HEADS, OUT)
    ai = (h4 * ati_ref[...][:HEADS][None, :, :]).sum(-1)
    aj = (h4 * atj_ref[...][:HEADS][None, :, :]).sum(-1)
    pad = jnp.zeros((xb.shape[0], 128 - 2 * HEADS), dtype=jnp.float32)
    aij_ref[...] = jnp.concatenate([ai, aj, pad], axis=1)


def _tc_project(x_pad, W, att_i8, att_j8):
    nb = 1024
    grid = N_TC // nb
    return pl.pallas_call(
        _tc_body,
        grid=(grid,),
        in_specs=[
            pl.BlockSpec((nb, IN), lambda i: (i, 0)),
            pl.BlockSpec((IN, HEADS * OUT), lambda i: (0, 0)),
            pl.BlockSpec((8, 128), lambda i: (0, 0)),
            pl.BlockSpec((8, 128), lambda i: (0, 0)),
        ],
        out_specs=[
            pl.BlockSpec((nb, HEADS * OUT), lambda i: (i, 0)),
            pl.BlockSpec((nb, 128), lambda i: (i, 0)),
        ],
        out_shape=[
            jax.ShapeDtypeStruct((N_TC, HEADS * OUT), jnp.float32),
            jax.ShapeDtypeStruct((N_TC, 128), jnp.float32),
        ],
    )(x_pad, W, att_i8, att_j8)


def _sc_body(hflat, ai_hbm, aj_hbm, m_hbm, rc_flat, zden, out_hbm, ex_hbm,
             den_sp, out_sp, sem, sem2, sem3, sem4):
    c = lax.axis_index("c")
    s = lax.axis_index("s")
    iota = lax.iota(jnp.int32, 16)

    def region_a(ai_t, aj_t, rc_p, st64, ri64, m_v):
        p = s // 2
        hl = s % 2
        hh = 2 * c + hl
        pltpu.sync_copy(ai_hbm.at[hh], ai_t)
        pltpu.sync_copy(aj_hbm.at[hh], aj_t)
        pltpu.sync_copy(m_hbm.at[c], m_v)
        pltpu.sync_copy(rc_flat.at[pl.ds(p * 2 * E_PT, 2 * E_PT)], rc_p)

        @pl.when(s < 10)
        def _():
            pltpu.sync_copy(zden, den_sp.at[pl.ds(s * 2000, 2000)])

        plsc.subcore_barrier()

        def pa_body(b, carry):
            for g in range(4):
                sl = pl.ds(g * 16, 16)
                rc = rc_p[pl.ds(b * BE + g * 16, 16)]
                r = lax.shift_right_logical(rc, 14)
                cc = lax.bitwise_and(rc, SHIFT - 1)
                gid = jnp.full((16,), p * 2 * E_PT + b * BE + g * 16,
                               jnp.int32) + iota
                live = gid < E_REAL
                air = plsc.load_gather(ai_t, [r])
                ajc = plsc.load_gather(aj_t, [cc])
                sv = air + ajc
                alpha = jnp.where(sv > 0, sv, NEG * sv)
                bb = air + m_v[pl.ds(hl * 16, 16)]
                bp = jnp.where(bb > 0, bb, NEG * bb)
                ex = jnp.where(live, jnp.exp(alpha - bp), 0.0)
                st64[sl] = ex
                ri64[sl] = r + jnp.full((16,), hl * N, jnp.int32)
            pltpu.sync_copy(st64, den_sp.at[ri64], add=True)
            pltpu.sync_copy(
                st64,
                ex_hbm.at[pl.ds(hh * E_PAD + p * 2 * E_PT + b * BE, BE)])
            return carry

        lax.fori_loop(0, NB_A, pa_body, 0)

    pl.run_scoped(
        region_a,
        ai_t=pltpu.VMEM((N,), jnp.float32),
        aj_t=pltpu.VMEM((N,), jnp.float32),
        rc_p=pltpu.VMEM((2 * E_PT,), jnp.int32),
        st64=pltpu.VMEM((BE,), jnp.float32),
        ri64=pltpu.VMEM((BE,), jnp.int32),
        m_v=pltpu.VMEM((32,), jnp.float32),
    )
    plsc.subcore_barrier()

    def region_b(rc_t, rvA, rvB, exA, exB, w64, d64,
                 ciA, riA, diA, ciB, riB, diB):
        pltpu.sync_copy(rc_flat.at[pl.ds(s * E_PT, E_PT)], rc_t)

        def zero_j(j, carry2):
            for q in range(8):
                rvA[j, pl.ds(q * 16, 16)] = jnp.zeros((16,), jnp.float32)
            return carry2

        bufs = ((rvA, exA, ciA, riA, diA, sem, sem2),
                (rvB, exB, ciB, riB, diB, sem3, sem4))

        for h in range(2):
            hh = 2 * c + h
            lax.fori_loop(0, BE, zero_j, 0)
            for k in range(9):
                pltpu.sync_copy(
                    rvA, out_sp.at[pl.ds(s * ROWS_PT + k * BE, BE)])
            pltpu.sync_copy(
                rvA.at[pl.ds(0, ROWS_PT - 9 * BE)],
                out_sp.at[pl.ds(s * ROWS_PT + 9 * BE, ROWS_PT - 9 * BE)])
            plsc.subcore_barrier()

            def issue(b, which, h=h, hh=hh):
                rv, ex_b, ci, ri, di, sr, se = bufs[which]
                for g in range(4):
                    sl = pl.ds(g * 16, 16)
                    rc = rc_t[pl.ds(b * BE + g * 16, 16)]
                    r = lax.shift_right_logical(rc, 14)
                    cc = lax.bitwise_and(rc, SHIFT - 1)
                    ci[sl] = cc + jnp.full((16,), hh * N, jnp.int32)
                    ri[sl] = r
                    di[sl] = r + jnp.full((16,), h * N, jnp.int32)
                pltpu.async_copy(hflat.at[ci], rv, sr)
                pltpu.async_copy(
                    ex_hbm.at[pl.ds(hh * E_PAD + s * E_PT + b * BE, BE)],
                    ex_b, se)

            def consume(which):
                rv, ex_b, ci, ri, di, sr, se = bufs[which]
                pltpu.sync_copy(den_sp.at[di], d64)
                pltpu.make_async_copy(
                    ex_hbm.at[pl.ds(0, BE)], ex_b, se).wait()
                for g in range(4):
                    sl = pl.ds(g * 16, 16)
                    w64[sl] = ex_b[sl] / (d64[sl] + 1e-10)
                pltpu.make_async_copy(hflat.at[ci], rv, sr).wait()

                def scale_j(j, carry2):
                    wv = plsc.load_gather(
                        w64, [jnp.full((16,), j, jnp.int32)])
                    for q in range(8):
                        rv[j, pl.ds(q * 16, 16)] = (
                            rv[j, pl.ds(q * 16, 16)] * wv)
                    return carry2

                lax.fori_loop(0, BE, scale_j, 0)
                pltpu.sync_copy(rv, out_sp.at[ri], add=True)

            issue(0, 0)

            def pb_body(b, carry):
                @pl.when(b % 2 == 0)
                def _():
                    @pl.when(b + 1 < NB)
                    def _():
                        issue(b + 1, 1)
                    consume(0)

                @pl.when(b % 2 == 1)
                def _():
                    @pl.when(b + 1 < NB)
                    def _():
                        issue(b + 1, 0)
                    consume(1)

                return carry

            lax.fori_loop(0, NB, pb_body, 0)
            plsc.subcore_barrier()
            pltpu.sync_copy(
                out_sp.at[pl.ds(s * ROWS_PT, ROWS_PT)],
                out_hbm.at[pl.ds(hh * N + s * ROWS_PT, ROWS_PT)])
            plsc.subcore_barrier()

    pl.run_scoped(
        region_b,
        rc_t=pltpu.VMEM((E_PT,), jnp.int32),
        rvA=pltpu.VMEM((BE, 128), jnp.float32),
        rvB=pltpu.VMEM((BE, 128), jnp.float32),
        exA=pltpu.VMEM((BE,), jnp.float32),
        exB=pltpu.VMEM((BE,), jnp.float32),
        w64=pltpu.VMEM((BE,), jnp.float32),
        d64=pltpu.VMEM((BE,), jnp.float32),
        ciA=pltpu.VMEM((BE,), jnp.int32),
        riA=pltpu.VMEM((BE,), jnp.int32),
        diA=pltpu.VMEM((BE,), jnp.int32),
        ciB=pltpu.VMEM((BE,), jnp.int32),
        riB=pltpu.VMEM((BE,), jnp.int32),
        diB=pltpu.VMEM((BE,), jnp.int32),
    )


def _sc_run(hflat, ai_T, aj_T, Msc, rc_flat, zden):
    mesh = plsc.VectorSubcoreMesh(core_axis_name="c", subcore_axis_name="s")
    kern = functools.partial(
        pl.kernel,
        mesh=mesh,
        compiler_params=pltpu.CompilerParams(
            needs_layout_passes=False, use_tc_tiling_on_sc=False),
        out_type=(
            jax.ShapeDtypeStruct((HEADS * N, 128), jnp.float32),
            jax.ShapeDtypeStruct((HEADS * E_PAD,), jnp.float32),
        ),
        scratch_types=[
            pltpu.VMEM_SHARED((2 * N,), jnp.float32),
            pltpu.VMEM_SHARED((N, 128), jnp.float32),
            pltpu.SemaphoreType.DMA,
            pltpu.SemaphoreType.DMA,
            pltpu.SemaphoreType.DMA,
            pltpu.SemaphoreType.DMA,
        ],
    )(_sc_body)
    return kern(hflat, ai_T, aj_T, Msc, rc_flat, zden)


@jax.jit
def kernel(x, edge_index, W, att, bias):
    x_pad = jnp.concatenate(
        [x, jnp.zeros((N_TC - N, IN), jnp.float32)], axis=0)
    att2 = att[0]
    att_i8 = jnp.zeros((8, 128), jnp.float32).at[:HEADS].set(att2[:, :OUT])
    att_j8 = jnp.zeros((8, 128), jnp.float32).at[:HEADS].set(att2[:, OUT:])

    h2d, aij = _tc_project(x_pad, W, att_i8, att_j8)
    h2d = h2d[:N]
    aij = aij[:N]

    hflat = h2d.reshape(N, HEADS, OUT).transpose(1, 0, 2) \
               .reshape(HEADS * N, 128)
    ai_T = aij[:, :HEADS].T
    aj_T = aij[:, HEADS:2 * HEADS].T
    M4 = jnp.max(aj_T, axis=1)
    Msc = jnp.broadcast_to(M4.reshape(2, 2, 1), (2, 2, 16)) \
             .reshape(2, 32).astype(jnp.float32)

    loops = jnp.arange(N, dtype=edge_index.dtype)
    ei = jnp.concatenate([edge_index, jnp.stack([loops, loops])], axis=1)
    rc = ei[0] * SHIFT + ei[1]
    rc_flat = jnp.concatenate(
        [rc, jnp.zeros((E_PAD - E_REAL,), rc.dtype)])

    zden = jnp.zeros((2000,), jnp.float32)

    out_flat, _ = _sc_run(hflat, ai_T, aj_T, Msc, rc_flat, zden)

    out = out_flat.reshape(HEADS, N, 128).transpose(1, 0, 2)
    return out.reshape(N, HEADS * OUT) + bias


# speedup vs baseline: 38.7117x; 1.0164x over previous
import functools

import jax
import jax.numpy as jnp
from jax import lax
from jax.experimental import pallas as pl
from jax.experimental.pallas import tpu as pltpu
from jax.experimental.pallas import tpu_sc as plsc

N = 10000
E = 160000
IN = 256
HEADS = 4
OUT = 128
NEG = 0.2

N_TC = 10240
BE = 64
NB = 167
E_PT = NB * BE
E_PAD = 16 * E_PT
NB_A = 2 * NB
E_REAL = E + N
ROWS_PT = N // 16
SHIFT = 16384


def _tc_body(x_ref, w_ref, ati_ref, atj_ref, h_ref, aij_ref):
    xb = x_ref[...]
    h = jnp.dot(xb, w_ref[...], preferred_element_type=jnp.float32)
    h_ref[...] = h
    h4 = h.reshape(xb.shape[0], HEADS, OUT)
    ai = (h4 * ati_ref[...][:HEADS][None, :, :]).sum(-1)
    aj = (h4 * atj_ref[...][:HEADS][None, :, :]).sum(-1)
    pad = jnp.zeros((xb.shape[0], 128 - 2 * HEADS), dtype=jnp.float32)
    aij_ref[...] = jnp.concatenate([ai, aj, pad], axis=1)


def _tc_project(x_pad, W, att_i8, att_j8):
    nb = 1024
    grid = N_TC // nb
    return pl.pallas_call(
        _tc_body,
        grid=(grid,),
        in_specs=[
            pl.BlockSpec((nb, IN), lambda i: (i, 0)),
            pl.BlockSpec((IN, HEADS * OUT), lambda i: (0, 0)),
            pl.BlockSpec((8, 128), lambda i: (0, 0)),
            pl.BlockSpec((8, 128), lambda i: (0, 0)),
        ],
        out_specs=[
            pl.BlockSpec((nb, HEADS * OUT), lambda i: (i, 0)),
            pl.BlockSpec((nb, 128), lambda i: (i, 0)),
        ],
        out_shape=[
            jax.ShapeDtypeStruct((N_TC, HEADS * OUT), jnp.float32),
            jax.ShapeDtypeStruct((N_TC, 128), jnp.float32),
        ],
    )(x_pad, W, att_i8, att_j8)


def _sc_body(hflat, ai_hbm, aj_hbm, m_hbm, rc_flat, zden, out_hbm, ex_hbm,
             den_sp, out_sp, sem, sem2, sem3, sem4):
    c = lax.axis_index("c")
    s = lax.axis_index("s")
    iota = lax.iota(jnp.int32, 16)

    def region_a(ai_t, aj_t, rc_p, st64, st64b, ri64, ri64b, m_v):
        p = s // 2
        hl = s % 2
        hh = 2 * c + hl
        pltpu.sync_copy(ai_hbm.at[hh], ai_t)
        pltpu.sync_copy(aj_hbm.at[hh], aj_t)
        pltpu.sync_copy(m_hbm.at[c], m_v)
        pltpu.sync_copy(rc_flat.at[pl.ds(p * 2 * E_PT, 2 * E_PT)], rc_p)

        @pl.when(s < 10)
        def _():
            pltpu.sync_copy(zden, den_sp.at[pl.ds(s * 2000, 2000)])

        plsc.subcore_barrier()

        abufs = ((st64, ri64, sem), (st64b, ri64b, sem2))

        def pa_work(b, which, p=p, hl=hl, hh=hh):
            st, ri, se = abufs[which]

            @pl.when(b >= 2)
            def _():
                pltpu.make_async_copy(
                    st, ex_hbm.at[pl.ds(0, BE)], se).wait()

            for g in range(4):
                sl = pl.ds(g * 16, 16)
                rc = rc_p[pl.ds(b * BE + g * 16, 16)]
                r = lax.shift_right_logical(rc, 14)
                cc = lax.bitwise_and(rc, SHIFT - 1)
                gid = jnp.full((16,), p * 2 * E_PT + b * BE + g * 16,
                               jnp.int32) + iota
                live = gid < E_REAL
                air = plsc.load_gather(ai_t, [r])
                ajc = plsc.load_gather(aj_t, [cc])
                sv = air + ajc
                alpha = jnp.where(sv > 0, sv, NEG * sv)
                bb = air + m_v[pl.ds(hl * 16, 16)]
                bp = jnp.where(bb > 0, bb, NEG * bb)
                ex = jnp.where(live, jnp.exp(alpha - bp), 0.0)
                st[sl] = ex
                ri[sl] = r + jnp.full((16,), hl * N, jnp.int32)
            pltpu.sync_copy(st, den_sp.at[ri], add=True)
            pltpu.async_copy(
                st,
                ex_hbm.at[pl.ds(hh * E_PAD + p * 2 * E_PT + b * BE, BE)],
                se)

        def pa_body(b, carry):
            @pl.when(b % 2 == 0)
            def _():
                pa_work(b, 0)

            @pl.when(b % 2 == 1)
            def _():
                pa_work(b, 1)

            return carry

        lax.fori_loop(0, NB_A, pa_body, 0)
        for which in range(2):
            st, ri, se = abufs[which]
            pltpu.make_async_copy(st, ex_hbm.at[pl.ds(0, BE)], se).wait()

    pl.run_scoped(
        region_a,
        ai_t=pltpu.VMEM((N,), jnp.float32),
        aj_t=pltpu.VMEM((N,), jnp.float32),
        rc_p=pltpu.VMEM((2 * E_PT,), jnp.int32),
        st64=pltpu.VMEM((BE,), jnp.float32),
        st64b=pltpu.VMEM((BE,), jnp.float32),
        ri64=pltpu.VMEM((BE,), jnp.int32),
        ri64b=pltpu.VMEM((BE,), jnp.int32),
        m_v=pltpu.VMEM((32,), jnp.float32),
    )
    plsc.subcore_barrier()

    def region_b(rc_t, rvA, rvB, exA, exB, w64, d64,
                 ciA, riA, diA, ciB, riB, diB):
        pltpu.sync_copy(rc_flat.at[pl.ds(s * E_PT, E_PT)], rc_t)

        def zero_j(j, carry2):
            for q in range(8):
                rvA[j, pl.ds(q * 16, 16)] = jnp.zeros((16,), jnp.float32)
            return carry2

        bufs = ((rvA, exA, ciA, riA, diA, sem, sem2),
                (rvB, exB, ciB, riB, diB, sem3, sem4))

        for h in range(2):
            hh = 2 * c + h
            lax.fori_loop(0, BE, zero_j, 0)
            for k in range(9):
                pltpu.sync_copy(
                    rvA, out_sp.at[pl.ds(s * ROWS_PT + k * BE, BE)])
            pltpu.sync_copy(
                rvA.at[pl.ds(0, ROWS_PT - 9 * BE)],
                out_sp.at[pl.ds(s * ROWS_PT + 9 * BE, ROWS_PT - 9 * BE)])
            plsc.subcore_barrier()

            def issue(b, which, h=h, hh=hh):
                rv, ex_b, ci, ri, di, sr, se = bufs[which]
                for g in range(4):
                    sl = pl.ds(g * 16, 16)
                    rc = rc_t[pl.ds(b * BE + g * 16, 16)]
                    r = lax.shift_right_logical(rc, 14)
                    cc = lax.bitwise_and(rc, SHIFT - 1)
                    ci[sl] = cc + jnp.full((16,), hh * N, jnp.int32)
                    ri[sl] = r
                    di[sl] = r + jnp.full((16,), h * N, jnp.int32)
                pltpu.async_copy(hflat.at[ci], rv, sr)
                pltpu.async_copy(
                    ex_hbm.at[pl.ds(hh * E_PAD + s * E_PT + b * BE, BE)],
                    ex_b, se)

            def consume(which):
                rv, ex_b, ci, ri, di, sr, se = bufs[which]
                pltpu.sync_copy(den_sp.at[di], d64)
                pltpu.make_async_copy(
                    ex_hbm.at[pl.ds(0, BE)], ex_b, se).wait()
                for g in range(4):
                    sl = pl.ds(g * 16, 16)
                    w64[sl] = ex_b[sl] / (d64[sl] + 1e-10)
                pltpu.make_async_copy(hflat.at[ci], rv, sr).wait()

                def scale_j(j, carry2):
                    wv = plsc.load_gather(
                        w64, [jnp.full((16,), j, jnp.int32)])
                    for q in range(8):
                        rv[j, pl.ds(q * 16, 16)] = (
                            rv[j, pl.ds(q * 16, 16)] * wv)
                    return carry2

                lax.fori_loop(0, BE, scale_j, 0)
                pltpu.sync_copy(rv, out_sp.at[ri], add=True)

            issue(0, 0)

            def pb_body(b, carry):
                @pl.when(b % 2 == 0)
                def _():
                    @pl.when(b + 1 < NB)
                    def _():
                        issue(b + 1, 1)
                    consume(0)

                @pl.when(b % 2 == 1)
                def _():
                    @pl.when(b + 1 < NB)
                    def _():
                        issue(b + 1, 0)
                    consume(1)

                return carry

            lax.fori_loop(0, NB, pb_body, 0)
            plsc.subcore_barrier()
            pltpu.sync_copy(
                out_sp.at[pl.ds(s * ROWS_PT, ROWS_PT)],
                out_hbm.at[pl.ds(hh * N + s * ROWS_PT, ROWS_PT)])
            plsc.subcore_barrier()

    pl.run_scoped(
        region_b,
        rc_t=pltpu.VMEM((E_PT,), jnp.int32),
        rvA=pltpu.VMEM((BE, 128), jnp.float32),
        rvB=pltpu.VMEM((BE, 128), jnp.float32),
        exA=pltpu.VMEM((BE,), jnp.float32),
        exB=pltpu.VMEM((BE,), jnp.float32),
        w64=pltpu.VMEM((BE,), jnp.float32),
        d64=pltpu.VMEM((BE,), jnp.float32),
        ciA=pltpu.VMEM((BE,), jnp.int32),
        riA=pltpu.VMEM((BE,), jnp.int32),
        diA=pltpu.VMEM((BE,), jnp.int32),
        ciB=pltpu.VMEM((BE,), jnp.int32),
        riB=pltpu.VMEM((BE,), jnp.int32),
        diB=pltpu.VMEM((BE,), jnp.int32),
    )


def _sc_run(hflat, ai_T, aj_T, Msc, rc_flat, zden):
    mesh = plsc.VectorSubcoreMesh(core_axis_name="c", subcore_axis_name="s")
    kern = functools.partial(
        pl.kernel,
        mesh=mesh,
        compiler_params=pltpu.CompilerParams(
            needs_layout_passes=False, use_tc_tiling_on_sc=False),
        out_type=(
            jax.ShapeDtypeStruct((HEADS * N, 128), jnp.float32),
            jax.ShapeDtypeStruct((HEADS * E_PAD,), jnp.float32),
        ),
        scratch_types=[
            pltpu.VMEM_SHARED((2 * N,), jnp.float32),
            pltpu.VMEM_SHARED((N, 128), jnp.float32),
            pltpu.SemaphoreType.DMA,
            pltpu.SemaphoreType.DMA,
            pltpu.SemaphoreType.DMA,
            pltpu.SemaphoreType.DMA,
        ],
    )(_sc_body)
    return kern(hflat, ai_T, aj_T, Msc, rc_flat, zden)


@jax.jit
def kernel(x, edge_index, W, att, bias):
    x_pad = jnp.concatenate(
        [x, jnp.zeros((N_TC - N, IN), jnp.float32)], axis=0)
    att2 = att[0]
    att_i8 = jnp.zeros((8, 128), jnp.float32).at[:HEADS].set(att2[:, :OUT])
    att_j8 = jnp.zeros((8, 128), jnp.float32).at[:HEADS].set(att2[:, OUT:])

    h2d, aij = _tc_project(x_pad, W, att_i8, att_j8)
    h2d = h2d[:N]
    aij = aij[:N]

    hflat = h2d.reshape(N, HEADS, OUT).transpose(1, 0, 2) \
               .reshape(HEADS * N, 128)
    ai_T = aij[:, :HEADS].T
    aj_T = aij[:, HEADS:2 * HEADS].T
    M4 = jnp.max(aj_T, axis=1)
    Msc = jnp.broadcast_to(M4.reshape(2, 2, 1), (2, 2, 16)) \
             .reshape(2, 32).astype(jnp.float32)

    loops = jnp.arange(N, dtype=edge_index.dtype)
    ei = jnp.concatenate([edge_index, jnp.stack([loops, loops])], axis=1)
    rc = ei[0] * SHIFT + ei[1]
    rc_flat = jnp.concatenate(
        [rc, jnp.zeros((E_PAD - E_REAL,), rc.dtype)])

    zden = jnp.zeros((2000,), jnp.float32)

    out_flat, _ = _sc_run(hflat, ai_T, aj_T, Msc, rc_flat, zden)

    out = out_flat.reshape(HEADS, N, 128).transpose(1, 0, 2)
    return out.reshape(N, HEADS * OUT) + bias
